# baseline (speedup 1.0000x reference)
import jax
import jax.numpy as jnp
from jax import lax
from jax.experimental import pallas as pl
from jax.experimental.pallas import tpu as pltpu
from jax.experimental.pallas import tpu_sc as plsc

N = 10000
E = 320000
D = 128
HALF = 64
NS = 16
L = 16
CH = 80
EPS = E // NS
NCH = EPS // CH
ROWS = N // NS
ZCH = 25


def _mp_body(h_hbm, efa_hbm, efb_hbm, src_hbm, dst_hbm, agga_hbm, aggb_hbm,
             h_sh, agg_sh,
             hr0, er0, pr0, idxs0, idxd0, sidx0,
             hr1, er1, pr1, idxs1, idxd1, sidx1, zbuf):
    c = lax.axis_index("c")
    s = lax.axis_index("s")
    col0 = c * HALF
    r0 = s * ROWS
    sets = ((hr0, er0, pr0, idxs0, idxd0, sidx0),
            (hr1, er1, pr1, idxs1, idxd1, sidx1))

    pltpu.sync_copy(h_hbm.at[pl.ds(r0, ROWS), pl.ds(col0, HALF)],
                    h_sh.at[pl.ds(r0, ROWS)])

    def _zrow(i, carry):
        z = jnp.zeros((L,), jnp.float32)
        for j in range(HALF // L):
            zbuf[i, pl.ds(j * L, L)] = z
        return carry

    def _zero_rows():
        def _zc(t, carry):
            pltpu.sync_copy(zbuf, agg_sh.at[pl.ds(r0 + t * ZCH, ZCH)])
            return carry
        lax.fori_loop(0, ROWS // ZCH, _zc, 0)

    lax.fori_loop(0, ZCH, _zrow, 0)
    _zero_rows()
    plsc.subcore_barrier()

    def _edge_phase(ef_hbm, gather_src, sidx_of):

        def _phase_inner(lh0, le0, lh1, le1, ss0, ss1):
            lsems = ((lh0, le0), (lh1, le1))
            ssems = (ss0, ss1)

            def _eb(k):
                return s * EPS + jnp.where(k < NCH, k, 0) * CH

            def _load_descs(k, st, ls):
                eb = _eb(k)
                d1 = pltpu.make_async_copy(h_sh.at[gather_src(st)],
                                           st[0], ls[0])
                d2 = pltpu.make_async_copy(
                    ef_hbm.at[pl.ds(eb, CH), pl.ds(col0, HALF)], st[1], ls[1])
                return d1, d2

            def _start_gathers(k, st, ls):
                eb = _eb(k)
                pltpu.sync_copy(src_hbm.at[pl.ds(eb, CH)], st[3])
                pltpu.sync_copy(dst_hbm.at[pl.ds(eb, CH)], st[4])
                d1, d2 = _load_descs(k, st, ls)
                d1.start()
                d2.start()

            _start_gathers(0, sets[0], lsems[0])
            _start_gathers(1, sets[1], lsems[1])

            def _scat_desc(st, b):
                return pltpu.make_async_copy(st[2], agg_sh.at[st[5]],
                                             ssems[b])

            @pl.loop(0, NCH, step=2)
            def _g(g):
                for b in (0, 1):
                    st = sets[b]
                    hr, er, pr = st[0], st[1], st[2]
                    k = g + b
                    d1, d2 = _load_descs(k, st, lsems[b])
                    d1.wait()
                    d2.wait()

                    @pl.when(k >= 2)
                    def _():
                        _scat_desc(st, b).wait()

                    @pl.loop(0, CH, unroll=4)
                    def _mul(i):
                        for v in range(HALF // L):
                            sl = pl.ds(v * L, L)
                            pr[i, sl] = er[i, sl] * hr[i, sl]
                    scat_src = sidx_of(st)
                    for v in range(CH // L):
                        sl = pl.ds(v * L, L)
                        st[5][sl] = scat_src[sl]
                    _scat_desc(st, b).start(add=True)
                    _start_gathers(k + 2, st, lsems[b])

            for b in (0, 1):
                d1, d2 = _load_descs(NCH + b, sets[b], lsems[b])
                d1.wait()
                d2.wait()
                _scat_desc(sets[b], b).wait()

        pl.run_scoped(_phase_inner, pltpu.SemaphoreType.DMA,
                      pltpu.SemaphoreType.DMA, pltpu.SemaphoreType.DMA,
                      pltpu.SemaphoreType.DMA, pltpu.SemaphoreType.DMA,
                      pltpu.SemaphoreType.DMA)

    def _writeback(out_hbm):
        pltpu.sync_copy(agg_sh.at[pl.ds(r0, ROWS)],
                        out_hbm.at[pl.ds(r0, ROWS), pl.ds(col0, HALF)])

    _edge_phase(efa_hbm, lambda st: st[3], lambda st: st[4])
    plsc.subcore_barrier()
    _writeback(agga_hbm)
    _zero_rows()
    plsc.subcore_barrier()
    _edge_phase(efb_hbm, lambda st: st[4], lambda st: st[3])
    plsc.subcore_barrier()
    _writeback(aggb_hbm)


_mp = pl.kernel(
    _mp_body,
    out_type=(jax.ShapeDtypeStruct((N, D), jnp.float32),
              jax.ShapeDtypeStruct((N, D), jnp.float32)),
    mesh=plsc.VectorSubcoreMesh(core_axis_name="c", subcore_axis_name="s"),
    compiler_params=pltpu.CompilerParams(use_tc_tiling_on_sc=False),
    scratch_types=[
        pltpu.VMEM_SHARED((N, HALF), jnp.float32),
        pltpu.VMEM_SHARED((N, HALF), jnp.float32),
        pltpu.VMEM((CH, HALF), jnp.float32),
        pltpu.VMEM((CH, HALF), jnp.float32),
        pltpu.VMEM((CH, HALF), jnp.float32),
        pltpu.VMEM((CH,), jnp.int32),
        pltpu.VMEM((CH,), jnp.int32),
        pltpu.VMEM((CH,), jnp.int32),
        pltpu.VMEM((CH, HALF), jnp.float32),
        pltpu.VMEM((CH, HALF), jnp.float32),
        pltpu.VMEM((CH, HALF), jnp.float32),
        pltpu.VMEM((CH,), jnp.int32),
        pltpu.VMEM((CH,), jnp.int32),
        pltpu.VMEM((CH,), jnp.int32),
        pltpu.VMEM((ZCH, HALF), jnp.float32),
    ],
)


def _dense_body(a_ref, b_ref, h_ref, w_ref, bias_ref, out_ref):
    w = w_ref[...]
    acc = jnp.dot(a_ref[...], w[:D, :], preferred_element_type=jnp.float32,
                  precision=lax.Precision.HIGHEST)
    acc = acc + jnp.dot(b_ref[...], w[D:, :], preferred_element_type=jnp.float32,
                        precision=lax.Precision.HIGHEST)
    acc = acc + bias_ref[...]
    out_ref[...] = h_ref[...] + jnp.maximum(acc, 0.0)


_RB = 1000

_dense = pl.pallas_call(
    _dense_body,
    out_shape=jax.ShapeDtypeStruct((N, D), jnp.float32),
    grid=(N // _RB,),
    in_specs=[
        pl.BlockSpec((_RB, D), lambda i: (i, 0)),
        pl.BlockSpec((_RB, D), lambda i: (i, 0)),
        pl.BlockSpec((_RB, D), lambda i: (i, 0)),
        pl.BlockSpec((2 * D, D), lambda i: (0, 0)),
        pl.BlockSpec((1, D), lambda i: (0, 0)),
    ],
    out_specs=pl.BlockSpec((_RB, D), lambda i: (i, 0)),
)


def kernel(x, edge_feat_a, edge_feat_b, W0, b0, W1, b1, edge_index):
    src = edge_index[0]
    dst = edge_index[1]
    h = x
    for (W, b) in ((W0, b0), (W1, b1)):
        agga, aggb = _mp(h, edge_feat_a, edge_feat_b, src, dst)
        h = _dense(agga, aggb, h, W, b.reshape(1, D))
    return h


# speedup vs baseline: 1.0648x; 1.0648x over previous
import jax
import jax.numpy as jnp
from jax import lax
from jax.experimental import pallas as pl
from jax.experimental.pallas import tpu as pltpu
from jax.experimental.pallas import tpu_sc as plsc

N = 10000
E = 320000
D = 128
HALF = 64
NS = 16
L = 16
CH = 80
EPS = E // NS
NCH = EPS // CH
ROWS = N // NS
ZCH = 25


def _mp_body(h_hbm, efa_hbm, efb_hbm, src_hbm, dst_hbm, agga_hbm, aggb_hbm,
             h_sh, agg_sh,
             hr0, er0, pr0, idxs0, idxd0, sidx0, eidx0,
             hr1, er1, pr1, idxs1, idxd1, sidx1, eidx1, seqb, zbuf):
    c = lax.axis_index("c")
    s = lax.axis_index("s")
    col0 = c * HALF
    r0 = s * ROWS
    sets = ((hr0, er0, pr0, idxs0, idxd0, sidx0, eidx0),
            (hr1, er1, pr1, idxs1, idxd1, sidx1, eidx1))

    for v in range(CH // L):
        seqb[pl.ds(v * L, L)] = lax.iota(jnp.int32, L) + (v * L)

    pltpu.sync_copy(h_hbm.at[pl.ds(r0, ROWS), pl.ds(col0, HALF)],
                    h_sh.at[pl.ds(r0, ROWS)])

    def _zrow(i, carry):
        z = jnp.zeros((L,), jnp.float32)
        for j in range(HALF // L):
            zbuf[i, pl.ds(j * L, L)] = z
        return carry

    def _zero_rows():
        def _zc(t, carry):
            pltpu.sync_copy(zbuf, agg_sh.at[pl.ds(r0 + t * ZCH, ZCH)])
            return carry
        lax.fori_loop(0, ROWS // ZCH, _zc, 0)

    lax.fori_loop(0, ZCH, _zrow, 0)
    _zero_rows()
    plsc.subcore_barrier()

    def _edge_phase(ef_hbm, gather_src, sidx_of):

        def _phase_inner(lh0, le0, lh1, le1, ss0, ss1):
            lsems = ((lh0, le0), (lh1, le1))
            ssems = (ss0, ss1)

            def _eb(k):
                return s * EPS + jnp.where(k < NCH, k, 0) * CH

            def _load_descs(k, st, ls):
                d1 = pltpu.make_async_copy(h_sh.at[gather_src(st)],
                                           st[0], ls[0])
                d2 = pltpu.make_async_copy(ef_hbm.at[st[6]], st[1], ls[1])
                return d1, d2

            def _start_gathers(k, st, ls):
                eb = _eb(k)
                pltpu.sync_copy(src_hbm.at[pl.ds(eb, CH)], st[3])
                pltpu.sync_copy(dst_hbm.at[pl.ds(eb, CH)], st[4])
                for v in range(CH // L):
                    sl = pl.ds(v * L, L)
                    st[6][sl] = seqb[sl] + eb
                d1, d2 = _load_descs(k, st, ls)
                d1.start()
                d2.start()

            _start_gathers(0, sets[0], lsems[0])
            _start_gathers(1, sets[1], lsems[1])

            def _scat_desc(st, b):
                return pltpu.make_async_copy(st[2], agg_sh.at[st[5]],
                                             ssems[b])

            @pl.loop(0, NCH, step=2)
            def _g(g):
                for b in (0, 1):
                    st = sets[b]
                    hr, er, pr = st[0], st[1], st[2]
                    k = g + b
                    d1, d2 = _load_descs(k, st, lsems[b])
                    d1.wait()
                    d2.wait()

                    @pl.when(k >= 2)
                    def _():
                        _scat_desc(st, b).wait()

                    def _mul(i, c2):
                        for v in range(HALF // L):
                            sl = pl.ds(v * L, L)
                            pr[i, sl] = er[i, pl.ds(col0 + v * L, L)] * hr[i, sl]
                        return c2
                    lax.fori_loop(0, CH, _mul, 0)
                    scat_src = sidx_of(st)
                    for v in range(CH // L):
                        sl = pl.ds(v * L, L)
                        st[5][sl] = scat_src[sl]
                    _scat_desc(st, b).start(add=True)
                    _start_gathers(k + 2, st, lsems[b])

            for b in (0, 1):
                d1, d2 = _load_descs(NCH + b, sets[b], lsems[b])
                d1.wait()
                d2.wait()
                _scat_desc(sets[b], b).wait()

        pl.run_scoped(_phase_inner, pltpu.SemaphoreType.DMA,
                      pltpu.SemaphoreType.DMA, pltpu.SemaphoreType.DMA,
                      pltpu.SemaphoreType.DMA, pltpu.SemaphoreType.DMA,
                      pltpu.SemaphoreType.DMA)

    def _writeback(out_hbm):
        pltpu.sync_copy(agg_sh.at[pl.ds(r0, ROWS)],
                        out_hbm.at[pl.ds(r0, ROWS), pl.ds(col0, HALF)])

    _edge_phase(efa_hbm, lambda st: st[3], lambda st: st[4])
    plsc.subcore_barrier()
    _writeback(agga_hbm)
    _zero_rows()
    plsc.subcore_barrier()
    _edge_phase(efb_hbm, lambda st: st[4], lambda st: st[3])
    plsc.subcore_barrier()
    _writeback(aggb_hbm)


_mp = pl.kernel(
    _mp_body,
    out_type=(jax.ShapeDtypeStruct((N, D), jnp.float32),
              jax.ShapeDtypeStruct((N, D), jnp.float32)),
    mesh=plsc.VectorSubcoreMesh(core_axis_name="c", subcore_axis_name="s"),
    compiler_params=pltpu.CompilerParams(use_tc_tiling_on_sc=False),
    scratch_types=[
        pltpu.VMEM_SHARED((N, HALF), jnp.float32),
        pltpu.VMEM_SHARED((N, HALF), jnp.float32),
        pltpu.VMEM((CH, HALF), jnp.float32),
        pltpu.VMEM((CH, D), jnp.float32),
        pltpu.VMEM((CH, HALF), jnp.float32),
        pltpu.VMEM((CH,), jnp.int32),
        pltpu.VMEM((CH,), jnp.int32),
        pltpu.VMEM((CH,), jnp.int32),
        pltpu.VMEM((CH,), jnp.int32),
        pltpu.VMEM((CH, HALF), jnp.float32),
        pltpu.VMEM((CH, D), jnp.float32),
        pltpu.VMEM((CH, HALF), jnp.float32),
        pltpu.VMEM((CH,), jnp.int32),
        pltpu.VMEM((CH,), jnp.int32),
        pltpu.VMEM((CH,), jnp.int32),
        pltpu.VMEM((CH,), jnp.int32),
        pltpu.VMEM((CH,), jnp.int32),
        pltpu.VMEM((ZCH, HALF), jnp.float32),
    ],
)


def _dense_body(a_ref, b_ref, h_ref, w_ref, bias_ref, out_ref):
    w = w_ref[...]
    acc = jnp.dot(a_ref[...], w[:D, :], preferred_element_type=jnp.float32,
                  precision=lax.Precision.HIGHEST)
    acc = acc + jnp.dot(b_ref[...], w[D:, :], preferred_element_type=jnp.float32,
                        precision=lax.Precision.HIGHEST)
    acc = acc + bias_ref[...]
    out_ref[...] = h_ref[...] + jnp.maximum(acc, 0.0)


_RB = 1000

_dense = pl.pallas_call(
    _dense_body,
    out_shape=jax.ShapeDtypeStruct((N, D), jnp.float32),
    grid=(N // _RB,),
    in_specs=[
        pl.BlockSpec((_RB, D), lambda i: (i, 0)),
        pl.BlockSpec((_RB, D), lambda i: (i, 0)),
        pl.BlockSpec((_RB, D), lambda i: (i, 0)),
        pl.BlockSpec((2 * D, D), lambda i: (0, 0)),
        pl.BlockSpec((1, D), lambda i: (0, 0)),
    ],
    out_specs=pl.BlockSpec((_RB, D), lambda i: (i, 0)),
)


def kernel(x, edge_feat_a, edge_feat_b, W0, b0, W1, b1, edge_index):
    src = edge_index[0]
    dst = edge_index[1]
    h = x
    for (W, b) in ((W0, b0), (W1, b1)):
        agga, aggb = _mp(h, edge_feat_a, edge_feat_b, src, dst)
        h = _dense(agga, aggb, h, W, b.reshape(1, D))
    return h


# speedup vs baseline: 3.6208x; 3.4004x over previous
import jax
import jax.numpy as jnp
from jax import lax
from jax.experimental import pallas as pl
from jax.experimental.pallas import tpu as pltpu
from jax.experimental.pallas import tpu_sc as plsc

N = 10000
E = 320000
D = 128
HALF = 64
NS = 16
L = 16
CH = 80
EPS = E // NS
NCH = EPS // CH
ROWS = N // NS
ZCH = 25


def _mp_body(h_hbm, efa_hbm, efb_hbm, src_hbm, dst_hbm, agga_hbm, aggb_hbm,
             h_sh, agg_sh,
             hr0, er0, pr0, idxs0, idxd0, sidx0, gs0, gd0,
             hr1, er1, pr1, idxs1, idxd1, sidx1, gs1, gd1, zbuf):
    c = lax.axis_index("c")
    s = lax.axis_index("s")
    col0 = c * HALF
    r0 = s * ROWS
    sets = ((hr0, er0, pr0, idxs0, idxd0, sidx0, gs0, gd0),
            (hr1, er1, pr1, idxs1, idxd1, sidx1, gs1, gd1))

    pltpu.sync_copy(h_hbm.at[pl.ds(r0, ROWS), pl.ds(col0, HALF)],
                    h_sh.at[pl.ds(r0, ROWS)])

    def _zrow(i, carry):
        z = jnp.zeros((L,), jnp.float32)
        for j in range(HALF // L):
            zbuf[i, pl.ds(j * L, L)] = z
        return carry

    def _zero_rows():
        def _zc(t, carry):
            pltpu.sync_copy(zbuf, agg_sh.at[pl.ds(r0 + t * ZCH, ZCH)])
            return carry
        lax.fori_loop(0, ROWS // ZCH, _zc, 0)

    lax.fori_loop(0, ZCH, _zrow, 0)
    _zero_rows()
    plsc.subcore_barrier()

    def _edge_phase(ef_hbm, gather_src, sidx_of):

        def _phase_inner(lh0, le0, lh1, le1, ss0, ss1, is0, is1):
            lsems = ((lh0, le0), (lh1, le1))
            ssems = (ss0, ss1)
            isems = (is0, is1)

            def _eb(k):
                return s * EPS + jnp.where(k < NCH, k, 0) * CH

            def _idx_descs(k, st, isem):
                eb = _eb(k)
                di1 = pltpu.make_async_copy(src_hbm.at[pl.ds(eb, CH)],
                                            st[3], isem)
                di2 = pltpu.make_async_copy(dst_hbm.at[pl.ds(eb, CH)],
                                            st[4], isem)
                return di1, di2

            def _start_idx(k, b):
                di1, di2 = _idx_descs(k, sets[b], isems[b])
                di1.start()
                di2.start()

            def _load_descs(k, st, ls):
                eb = _eb(k)
                d1 = pltpu.make_async_copy(h_sh.at[gather_src(st)],
                                           st[0], ls[0])
                d2 = pltpu.make_async_copy(
                    ef_hbm.at[pl.ds(eb, CH), pl.ds(col0, HALF)], st[1],
                    ls[1])
                return d1, d2

            def _advance(k, b):
                st = sets[b]
                di1, di2 = _idx_descs(k, st, isems[b])
                di1.wait()
                di2.wait()
                for v in range(CH // L):
                    sl = pl.ds(v * L, L)
                    st[6][sl] = st[3][sl]
                    st[7][sl] = st[4][sl]
                d1, d2 = _load_descs(k, st, lsems[b])
                d1.start()
                d2.start()
                _start_idx(k + 2, b)

            def _scat_desc(st, b):
                return pltpu.make_async_copy(st[2], agg_sh.at[st[5]],
                                             ssems[b])

            _start_idx(0, 0)
            _start_idx(1, 1)
            _advance(0, 0)
            _advance(1, 1)

            @pl.loop(0, NCH, step=2)
            def _g(g):
                for b in (0, 1):
                    st = sets[b]
                    hr, er, pr = st[0], st[1], st[2]
                    k = g + b
                    d1, d2 = _load_descs(k, st, lsems[b])
                    d1.wait()
                    d2.wait()

                    @pl.when(k >= 2)
                    def _():
                        _scat_desc(st, b).wait()

                    def _mul(i, c2):
                        for v in range(HALF // L):
                            sl = pl.ds(v * L, L)
                            pr[i, sl] = er[i, sl] * hr[i, sl]
                        return c2
                    lax.fori_loop(0, CH, _mul, 0)
                    scat_src = sidx_of(st)
                    for v in range(CH // L):
                        sl = pl.ds(v * L, L)
                        st[5][sl] = scat_src[sl]
                    _scat_desc(st, b).start(add=True)
                    _advance(k + 2, b)

            for b in (0, 1):
                d1, d2 = _load_descs(NCH + b, sets[b], lsems[b])
                d1.wait()
                d2.wait()
                di1, di2 = _idx_descs(NCH + 2 + b, sets[b], isems[b])
                di1.wait()
                di2.wait()
                _scat_desc(sets[b], b).wait()

        pl.run_scoped(_phase_inner, pltpu.SemaphoreType.DMA,
                      pltpu.SemaphoreType.DMA, pltpu.SemaphoreType.DMA,
                      pltpu.SemaphoreType.DMA, pltpu.SemaphoreType.DMA,
                      pltpu.SemaphoreType.DMA, pltpu.SemaphoreType.DMA,
                      pltpu.SemaphoreType.DMA)

    def _writeback(out_hbm):
        pltpu.sync_copy(agg_sh.at[pl.ds(r0, ROWS)],
                        out_hbm.at[pl.ds(r0, ROWS), pl.ds(col0, HALF)])

    _edge_phase(efa_hbm, lambda st: st[6], lambda st: st[7])
    plsc.subcore_barrier()
    _writeback(agga_hbm)
    _zero_rows()
    plsc.subcore_barrier()
    _edge_phase(efb_hbm, lambda st: st[7], lambda st: st[6])
    plsc.subcore_barrier()
    _writeback(aggb_hbm)


_mp = pl.kernel(
    _mp_body,
    out_type=(jax.ShapeDtypeStruct((N, D), jnp.float32),
              jax.ShapeDtypeStruct((N, D), jnp.float32)),
    mesh=plsc.VectorSubcoreMesh(core_axis_name="c", subcore_axis_name="s"),
    compiler_params=pltpu.CompilerParams(use_tc_tiling_on_sc=False),
    scratch_types=[
        pltpu.VMEM_SHARED((N, HALF), jnp.float32),
        pltpu.VMEM_SHARED((N, HALF), jnp.float32),
        pltpu.VMEM((CH, HALF), jnp.float32),
        pltpu.VMEM((CH, HALF), jnp.float32),
        pltpu.VMEM((CH, HALF), jnp.float32),
        pltpu.VMEM((CH,), jnp.int32),
        pltpu.VMEM((CH,), jnp.int32),
        pltpu.VMEM((CH,), jnp.int32),
        pltpu.VMEM((CH,), jnp.int32),
        pltpu.VMEM((CH,), jnp.int32),
        pltpu.VMEM((CH, HALF), jnp.float32),
        pltpu.VMEM((CH, HALF), jnp.float32),
        pltpu.VMEM((CH, HALF), jnp.float32),
        pltpu.VMEM((CH,), jnp.int32),
        pltpu.VMEM((CH,), jnp.int32),
        pltpu.VMEM((CH,), jnp.int32),
        pltpu.VMEM((CH,), jnp.int32),
        pltpu.VMEM((CH,), jnp.int32),
        pltpu.VMEM((ZCH, HALF), jnp.float32),
    ],
)


def _dense_body(a_ref, b_ref, h_ref, w_ref, bias_ref, out_ref):
    w = w_ref[...]
    acc = jnp.dot(a_ref[...], w[:D, :], preferred_element_type=jnp.float32,
                  precision=lax.Precision.HIGHEST)
    acc = acc + jnp.dot(b_ref[...], w[D:, :], preferred_element_type=jnp.float32,
                        precision=lax.Precision.HIGHEST)
    acc = acc + bias_ref[...]
    out_ref[...] = h_ref[...] + jnp.maximum(acc, 0.0)


_RB = 1000

_dense = pl.pallas_call(
    _dense_body,
    out_shape=jax.ShapeDtypeStruct((N, D), jnp.float32),
    grid=(N // _RB,),
    in_specs=[
        pl.BlockSpec((_RB, D), lambda i: (i, 0)),
        pl.BlockSpec((_RB, D), lambda i: (i, 0)),
        pl.BlockSpec((_RB, D), lambda i: (i, 0)),
        pl.BlockSpec((2 * D, D), lambda i: (0, 0)),
        pl.BlockSpec((1, D), lambda i: (0, 0)),
    ],
    out_specs=pl.BlockSpec((_RB, D), lambda i: (i, 0)),
)


def kernel(x, edge_feat_a, edge_feat_b, W0, b0, W1, b1, edge_index):
    src = edge_index[0]
    dst = edge_index[1]
    h = x
    for (W, b) in ((W0, b0), (W1, b1)):
        agga, aggb = _mp(h, edge_feat_a, edge_feat_b, src, dst)
        h = _dense(agga, aggb, h, W, b.reshape(1, D))
    return h
